# baseline (speedup 1.0000x reference)
import jax
import jax.numpy as jnp
from jax import lax
from jax.experimental import pallas as pl
from jax.experimental.pallas import tpu as pltpu
from jax.experimental.pallas import tpu_sc as plsc

NC = 2
NS = 16
NW = NC * NS

CHUNK = 128
NCHUNK = 126
PER_W = CHUNK * NCHUNK
PAD_TOTAL = NW * PER_W
GROUPS = PER_W // 16

COVER = 999936
NTILE = COVER // 128
MAIN_TILES = 8 * NTILE
TILES_PW = MAIN_TILES // NW
CORNER_BASE = MAIN_TILES * 1024
CORNER_TILES = 4

LN2 = 0.6931471805599453
_C = (0.01477872, -0.18029977, 0.9618663, -2.9452062,
      5.7234015, -7.443873, 7.1100354, -3.2407022)


def _compact_body(probs_t, corner, flat4, corner_v, sem, csem):
    wid = lax.axis_index("s") * NC + lax.axis_index("c")
    g0 = wid * TILES_PW

    def fire(t, _):
        g = g0 + t
        band = g // NTILE
        st = g - band * NTILE
        pltpu.async_copy(
            probs_t.at[pl.ds(band * 8, 8), pl.ds(st * 128, 128)],
            flat4.at[g], sem)
        return 0
    lax.fori_loop(0, TILES_PW, fire, 0)

    @pl.when(wid == 0)
    def _():
        pltpu.sync_copy(corner, corner_v)
        pltpu.sync_copy(corner_v, flat4.at[pl.ds(MAIN_TILES, CORNER_TILES)])

    def drain(t, _):
        g = g0 + t
        band = g // NTILE
        st = g - band * NTILE
        pltpu.make_async_copy(
            probs_t.at[pl.ds(band * 8, 8), pl.ds(st * 128, 128)],
            flat4.at[g], sem).wait()
        return 0
    lax.fori_loop(0, TILES_PW, drain, 0)


def _make_compact():
    mesh = plsc.VectorSubcoreMesh(
        core_axis_name="c", subcore_axis_name="s",
        num_cores=NC, num_subcores=NS)
    return pl.kernel(
        _compact_body,
        out_type=jax.ShapeDtypeStruct(
            (MAIN_TILES + CORNER_TILES, 8, 128), jnp.float32),
        mesh=mesh,
        scratch_types=[
            pltpu.VMEM((CORNER_TILES, 8, 128), jnp.float32),
            pltpu.SemaphoreType.DMA,
            pltpu.SemaphoreType.DMA,
        ],
        compiler_params=pltpu.CompilerParams(use_tc_tiling_on_sc=True),
    )


def _sc_body(probs, states, choices, weights, ll_out, w_out,
             st_v, ch_v, wt_v, vals_v, acc_v, sem_in, gsem):
    wid = lax.axis_index("s") * NC + lax.axis_index("c")
    base = wid * PER_W

    cp_st = pltpu.async_copy(states.at[pl.ds(base, PER_W)], st_v, sem_in)
    cp_ch = pltpu.async_copy(choices.at[pl.ds(base, PER_W)], ch_v, sem_in)
    cp_wt = pltpu.async_copy(weights.at[pl.ds(base, PER_W)], wt_v, sem_in)
    cp_st.wait()
    cp_ch.wait()
    cp_wt.wait()

    def idx_body(g, _):
        s = st_v[pl.ds(g * 16, 16)]
        c = ch_v[pl.ds(g * 16, 16)]
        main = (((c >> 3) * NTILE + (s >> 7)) << 10) + ((c & 7) << 7) \
            + (s & 127)
        tail = CORNER_BASE + (c << 6) + (s - COVER)
        st_v[pl.ds(g * 16, 16)] = jnp.where(s < COVER, main, tail)
        return 0
    lax.fori_loop(0, GROUPS, idx_body, 0, unroll=8)

    def fire_body(j, _):
        pltpu.async_copy(
            probs.at[st_v.at[pl.ds(j * CHUNK, CHUNK)]],
            vals_v.at[pl.ds(j * CHUNK, CHUNK)], gsem)
        return 0
    lax.fori_loop(0, NCHUNK, fire_body, 0)

    def comp_body(j, carry):
        acc_ll, acc_w = carry
        pltpu.make_async_copy(
            probs.at[st_v.at[pl.ds(j * CHUNK, CHUNK)]],
            vals_v.at[pl.ds(j * CHUNK, CHUNK)], gsem).wait()
        cbase = j * CHUNK
        for g in range(CHUNK // 16):
            p = vals_v[pl.ds(cbase + g * 16, 16)]
            w = wt_v[pl.ds(cbase + g * 16, 16)]
            p = jnp.minimum(jnp.maximum(p, jnp.float32(1e-10)),
                            jnp.float32(1.0))
            bits = lax.bitcast_convert_type(p, jnp.int32)
            e = (bits >> 23) - 127
            f = lax.bitcast_convert_type(
                (bits & 0x7FFFFF) | 0x3F800000, jnp.float32)
            poly = jnp.float32(_C[0])
            for c in _C[1:]:
                poly = poly * f + jnp.float32(c)
            acc_ll = acc_ll + (e.astype(jnp.float32) + poly) * w
            acc_w = acc_w + w
        return acc_ll, acc_w

    zeros = jnp.zeros((16,), jnp.float32)
    acc_ll, acc_w = lax.fori_loop(0, NCHUNK, comp_body, (zeros, zeros))

    acc_v[...] = acc_ll
    pltpu.sync_copy(acc_v, ll_out.at[wid])
    acc_v[...] = acc_w
    pltpu.sync_copy(acc_v, w_out.at[wid])


def _make_sc_call():
    mesh = plsc.VectorSubcoreMesh(
        core_axis_name="c", subcore_axis_name="s",
        num_cores=NC, num_subcores=NS)
    return pl.kernel(
        _sc_body,
        out_type=[
            jax.ShapeDtypeStruct((NW, 16), jnp.float32),
            jax.ShapeDtypeStruct((NW, 16), jnp.float32),
        ],
        mesh=mesh,
        scratch_types=[
            pltpu.VMEM((PER_W,), jnp.int32),
            pltpu.VMEM((PER_W,), jnp.int32),
            pltpu.VMEM((PER_W,), jnp.float32),
            pltpu.VMEM((PER_W,), jnp.float32),
            pltpu.VMEM((16,), jnp.float32),
            pltpu.SemaphoreType.DMA,
            pltpu.SemaphoreType.DMA,
        ],
    )


def kernel(choice_probs, obs_states, obs_choices, obs_weights):
    n_states, n_choices = choice_probs.shape
    n_obs = obs_states.shape[0]
    npad = PAD_TOTAL - n_obs
    pad_states = jnp.arange(npad, dtype=jnp.int32) % n_states
    states_p = jnp.concatenate([obs_states.astype(jnp.int32), pad_states])
    choices_p = jnp.concatenate(
        [obs_choices.astype(jnp.int32), jnp.zeros((npad,), jnp.int32)])
    weights_p = jnp.concatenate(
        [obs_weights, jnp.zeros((npad,), jnp.float32)])

    corner = choice_probs[COVER:, :].T.reshape(CORNER_TILES, 8, 128)
    flat4 = _make_compact()(choice_probs.T, corner)
    probs_flat = flat4.reshape(-1)

    ll_parts, w_parts = _make_sc_call()(
        probs_flat, states_p, choices_p, weights_p)

    ll = jnp.sum(ll_parts) * jnp.float32(LN2)
    sw = jnp.sum(w_parts)
    nll = -(ll / sw)
    return jnp.where(jnp.isfinite(nll), nll,
                     jnp.array(1e10, dtype=nll.dtype))


# speedup vs baseline: 31.7982x; 31.7982x over previous
import jax
import jax.numpy as jnp
from jax import lax
from jax.experimental import pallas as pl
from jax.experimental.pallas import tpu as pltpu
from jax.experimental.pallas import tpu_sc as plsc

NC = 2
NS = 16
NW = NC * NS

CHUNK = 128
NCHUNK = 126
PER_W = CHUNK * NCHUNK
PAD_TOTAL = NW * PER_W
GROUPS = PER_W // 16

COVER = 999936
NTILE = COVER // 128
MAIN_TILES = 8 * NTILE
TILES_PW = MAIN_TILES // NW
CORNER_BASE = MAIN_TILES * 1024
CORNER_TILES = 4
KT = 31
NSLAB = TILES_PW // KT

LN2 = 0.6931471805599453
_C = (0.01477872, -0.18029977, 0.9618663, -2.9452062,
      5.7234015, -7.443873, 7.1100354, -3.2407022)


def _compact_body(probs_t, corner, flat4, buf0, buf1, corner_v,
                  rsem0, rsem1, wsem0, wsem1):
    wid = lax.axis_index("s") * NC + lax.axis_index("c")
    band = wid // 4
    g0 = wid * TILES_PW
    tc0 = (wid - band * 4) * TILES_PW
    bufs = (buf0, buf1)
    rsems = (rsem0, rsem1)
    wsems = (wsem0, wsem1)

    def fire_reads(s, buf, rsem):
        for t in range(KT):
            tc = tc0 + s * KT + t
            pltpu.async_copy(
                probs_t.at[pl.ds(band * 8, 8), pl.ds(tc * 128, 128)],
                buf.at[t], rsem)

    def drain_reads(s, buf, rsem):
        for t in range(KT):
            tc = tc0 + s * KT + t
            pltpu.make_async_copy(
                probs_t.at[pl.ds(band * 8, 8), pl.ds(tc * 128, 128)],
                buf.at[t], rsem).wait()

    def write_slab(s, buf, wsem):
        pltpu.async_copy(buf, flat4.at[pl.ds(g0 + s * KT, KT)], wsem)
        pltpu.make_async_copy(
            buf, flat4.at[pl.ds(g0 + s * KT, KT)], wsem).wait()

    fire_reads(0, buf0, rsem0)
    fire_reads(1, buf1, rsem1)

    @pl.when(wid == 0)
    def _():
        pltpu.sync_copy(corner, corner_v)
        pltpu.sync_copy(corner_v, flat4.at[pl.ds(MAIN_TILES, CORNER_TILES)])

    def outer(jj, _):
        for b in range(2):
            s = jj * 2 + b

            @pl.when(s < NSLAB)
            def _():
                drain_reads(s, bufs[b], rsems[b])
                write_slab(s, bufs[b], wsems[b])

                @pl.when(s + 2 < NSLAB)
                def _():
                    fire_reads(s + 2, bufs[b], rsems[b])
        return 0

    lax.fori_loop(0, (NSLAB + 1) // 2, outer, 0)


def _make_compact():
    mesh = plsc.VectorSubcoreMesh(
        core_axis_name="c", subcore_axis_name="s",
        num_cores=NC, num_subcores=NS)
    return pl.kernel(
        _compact_body,
        out_type=jax.ShapeDtypeStruct(
            (MAIN_TILES + CORNER_TILES, 8, 128), jnp.float32),
        mesh=mesh,
        scratch_types=[
            pltpu.VMEM((KT, 8, 128), jnp.float32),
            pltpu.VMEM((KT, 8, 128), jnp.float32),
            pltpu.VMEM((CORNER_TILES, 8, 128), jnp.float32),
            pltpu.SemaphoreType.DMA,
            pltpu.SemaphoreType.DMA,
            pltpu.SemaphoreType.DMA,
            pltpu.SemaphoreType.DMA,
        ],
        compiler_params=pltpu.CompilerParams(use_tc_tiling_on_sc=True),
    )


def _sc_body(probs, states, choices, weights, ll_out, w_out,
             st_v, ch_v, wt_v, vals_v, acc_v, sem_in, gsem):
    wid = lax.axis_index("s") * NC + lax.axis_index("c")
    base = wid * PER_W

    cp_st = pltpu.async_copy(states.at[pl.ds(base, PER_W)], st_v, sem_in)
    cp_ch = pltpu.async_copy(choices.at[pl.ds(base, PER_W)], ch_v, sem_in)
    cp_wt = pltpu.async_copy(weights.at[pl.ds(base, PER_W)], wt_v, sem_in)
    cp_st.wait()
    cp_ch.wait()
    cp_wt.wait()

    def idx_body(g, _):
        s = st_v[pl.ds(g * 16, 16)]
        c = ch_v[pl.ds(g * 16, 16)]
        main = (((c >> 3) * NTILE + (s >> 7)) << 10) + ((c & 7) << 7) \
            + (s & 127)
        tail = CORNER_BASE + (c << 6) + (s - COVER)
        st_v[pl.ds(g * 16, 16)] = jnp.where(s < COVER, main, tail)
        return 0
    lax.fori_loop(0, GROUPS, idx_body, 0, unroll=8)

    def fire_body(j, _):
        pltpu.async_copy(
            probs.at[st_v.at[pl.ds(j * CHUNK, CHUNK)]],
            vals_v.at[pl.ds(j * CHUNK, CHUNK)], gsem)
        return 0
    lax.fori_loop(0, NCHUNK, fire_body, 0)

    def comp_body(j, carry):
        acc_ll, acc_w = carry
        pltpu.make_async_copy(
            probs.at[st_v.at[pl.ds(j * CHUNK, CHUNK)]],
            vals_v.at[pl.ds(j * CHUNK, CHUNK)], gsem).wait()
        cbase = j * CHUNK
        for g in range(CHUNK // 16):
            p = vals_v[pl.ds(cbase + g * 16, 16)]
            w = wt_v[pl.ds(cbase + g * 16, 16)]
            p = jnp.minimum(jnp.maximum(p, jnp.float32(1e-10)),
                            jnp.float32(1.0))
            bits = lax.bitcast_convert_type(p, jnp.int32)
            e = (bits >> 23) - 127
            f = lax.bitcast_convert_type(
                (bits & 0x7FFFFF) | 0x3F800000, jnp.float32)
            poly = jnp.float32(_C[0])
            for c in _C[1:]:
                poly = poly * f + jnp.float32(c)
            acc_ll = acc_ll + (e.astype(jnp.float32) + poly) * w
            acc_w = acc_w + w
        return acc_ll, acc_w

    zeros = jnp.zeros((16,), jnp.float32)
    acc_ll, acc_w = lax.fori_loop(0, NCHUNK, comp_body, (zeros, zeros))

    acc_v[...] = acc_ll
    pltpu.sync_copy(acc_v, ll_out.at[wid])
    acc_v[...] = acc_w
    pltpu.sync_copy(acc_v, w_out.at[wid])


def _make_sc_call():
    mesh = plsc.VectorSubcoreMesh(
        core_axis_name="c", subcore_axis_name="s",
        num_cores=NC, num_subcores=NS)
    return pl.kernel(
        _sc_body,
        out_type=[
            jax.ShapeDtypeStruct((NW, 16), jnp.float32),
            jax.ShapeDtypeStruct((NW, 16), jnp.float32),
        ],
        mesh=mesh,
        scratch_types=[
            pltpu.VMEM((PER_W,), jnp.int32),
            pltpu.VMEM((PER_W,), jnp.int32),
            pltpu.VMEM((PER_W,), jnp.float32),
            pltpu.VMEM((PER_W,), jnp.float32),
            pltpu.VMEM((16,), jnp.float32),
            pltpu.SemaphoreType.DMA,
            pltpu.SemaphoreType.DMA,
        ],
    )


def kernel(choice_probs, obs_states, obs_choices, obs_weights):
    n_states, n_choices = choice_probs.shape
    n_obs = obs_states.shape[0]
    npad = PAD_TOTAL - n_obs
    pad_states = jnp.arange(npad, dtype=jnp.int32) % n_states
    states_p = jnp.concatenate([obs_states.astype(jnp.int32), pad_states])
    choices_p = jnp.concatenate(
        [obs_choices.astype(jnp.int32), jnp.zeros((npad,), jnp.int32)])
    weights_p = jnp.concatenate(
        [obs_weights, jnp.zeros((npad,), jnp.float32)])

    corner = choice_probs[COVER:, :].T.reshape(CORNER_TILES, 8, 128)
    flat4 = _make_compact()(choice_probs.T, corner)
    probs_flat = flat4.reshape(-1)

    ll_parts, w_parts = _make_sc_call()(
        probs_flat, states_p, choices_p, weights_p)

    ll = jnp.sum(ll_parts) * jnp.float32(LN2)
    sw = jnp.sum(w_parts)
    nll = -(ll / sw)
    return jnp.where(jnp.isfinite(nll), nll,
                     jnp.array(1e10, dtype=nll.dtype))
